# baseline (speedup 1.0000x reference)
import functools

import jax
import jax.numpy as jnp
from jax import lax
from jax.experimental import pallas as pl
from jax.experimental.pallas import tpu as pltpu
from jax.experimental.pallas import tpu_sc as plsc

N = 10000
E = 320000
HID = 128
NPAD = 10240
NC = 2
NS = 16
NW = NC * NS
CB = 64
NHALF = 2
EH = E // NHALF
KSLOT = 4
ROWS_PER_SUB = NPAD // NS
EBLK = 3200
NBLK = 1000

_f32 = jnp.float32


def _silu(v):
    return v * jax.nn.sigmoid(v)


def _iota16():
    return lax.iota(jnp.int32, 16)


def _sc_gather(row, col, a_t, b_t, xn1d):
    ne = row.shape[0]
    nchunks = ne // CB
    cpw = -(-nchunks // NW)
    mesh = plsc.VectorSubcoreMesh(core_axis_name="c", subcore_axis_name="s")
    nsteps = cpw + 2
    nu = -(-nsteps // KSLOT)

    @functools.partial(
        pl.kernel,
        mesh=mesh,
        out_type=[jax.ShapeDtypeStruct((ne, HID), _f32),
                  jax.ShapeDtypeStruct((ne, HID), _f32),
                  jax.ShapeDtypeStruct((ne, HID), _f32)],
        scratch_types=([pltpu.VMEM((CB,), jnp.int32)] * (2 * KSLOT)
                       + [pltpu.VMEM((CB, HID), _f32)] * (2 * KSLOT)
                       + [pltpu.VMEM((CB, 16), _f32)] * KSLOT
                       + [pltpu.VMEM((3 * N,), _f32)]
                       + [pltpu.SemaphoreType.DMA] * (3 * KSLOT + 1)),
        compiler_params=pltpu.CompilerParams(needs_layout_passes=False, use_tc_tiling_on_sc=False),
    )
    def k(row_h, col_h, a_h, b_h, xn_h, ga_h, gb_h, r16_h, *scr):
        idxa = scr[0:KSLOT]
        idxb = scr[KSLOT:2 * KSLOT]
        bufa = scr[2 * KSLOT:3 * KSLOT]
        bufb = scr[3 * KSLOT:4 * KSLOT]
        rbuf = scr[4 * KSLOT:5 * KSLOT]
        xn_v = scr[5 * KSLOT]
        i_sem = scr[5 * KSLOT + 1:6 * KSLOT + 1]
        g_sem = scr[6 * KSLOT + 1:7 * KSLOT + 1]
        w_sem = scr[7 * KSLOT + 1:8 * KSLOT + 1]
        x_sem = scr[8 * KSLOT + 1]
        wid = lax.axis_index("s") * NC + lax.axis_index("c")

        pltpu.async_copy(xn_h, xn_v, x_sem).wait()

        def guard(t):
            return (t >= 0) & (wid + t * NW < nchunks)

        def chunk_off(t):
            return (wid + t * NW) * CB

        def issue_idx(t, z):
            @pl.when(guard(t))
            def _():
                off = chunk_off(t)
                pltpu.async_copy(row_h.at[pl.ds(off, CB)], idxa[z], i_sem[z])
                pltpu.async_copy(col_h.at[pl.ds(off, CB)], idxb[z], i_sem[z])

        def start_gather(t, z):
            @pl.when(guard(t))
            def _():
                pltpu.make_async_copy(row_h.at[pl.ds(0, CB)], idxa[z],
                                      i_sem[z]).wait()
                pltpu.make_async_copy(col_h.at[pl.ds(0, CB)], idxb[z],
                                      i_sem[z]).wait()
                pltpu.async_copy(a_h.at[idxa[z]], bufa[z], g_sem[z])
                pltpu.async_copy(b_h.at[idxb[z]], bufb[z], g_sem[z])
                i16 = _iota16()
                lane0 = jnp.zeros((16,), jnp.int32)
                for g in range(CB // 16):
                    rv = idxa[z][pl.ds(g * 16, 16)]
                    cv = idxb[z][pl.ds(g * 16, 16)]
                    r4 = rv * 3
                    c4 = cv * 3
                    rad = None
                    for comp in range(3):
                        xr = plsc.load_gather(xn_v, [r4 + comp])
                        xc = plsc.load_gather(xn_v, [c4 + comp])
                        dx = xr - xc
                        plsc.store_scatter(rbuf[z], [g * 16 + i16,
                                                     lane0 + 1 + comp], dx)
                        rad = dx * dx if rad is None else rad + dx * dx
                    plsc.store_scatter(rbuf[z], [g * 16 + i16, lane0], rad)

        def start_write(t, z):
            @pl.when(guard(t))
            def _():
                off = chunk_off(t)
                pltpu.make_async_copy(a_h.at[idxa[z]], bufa[z],
                                      g_sem[z]).wait()
                pltpu.make_async_copy(b_h.at[idxb[z]], bufb[z],
                                      g_sem[z]).wait()
                pltpu.async_copy(bufa[z], ga_h.at[pl.ds(off, CB)], w_sem[z])
                pltpu.async_copy(bufb[z], gb_h.at[pl.ds(off, CB)], w_sem[z])
                pltpu.async_copy(rbuf[z],
                                 r16_h.at[pl.ds(off, CB), pl.ds(0, 16)],
                                 w_sem[z])

        def wait_write(t, z):
            @pl.when(guard(t))
            def _():
                off = chunk_off(t)
                pltpu.make_async_copy(bufa[z], ga_h.at[pl.ds(off, CB)],
                                      w_sem[z]).wait()
                pltpu.make_async_copy(bufb[z], gb_h.at[pl.ds(off, CB)],
                                      w_sem[z]).wait()
                pltpu.make_async_copy(rbuf[z],
                                      r16_h.at[pl.ds(off, CB),
                                               pl.ds(0, 16)],
                                      w_sem[z]).wait()

        def body(u, carry):
            for z in range(KSLOT):
                t = u * KSLOT + z
                wait_write(t - KSLOT, z)
                issue_idx(t, z)
                start_gather(t - 1, (z - 1) % KSLOT)
                start_write(t - 2, (z - 2) % KSLOT)
            return carry

        lax.fori_loop(0, nu, body, 0)
        for c in range(max(nu * KSLOT - KSLOT, 0), cpw):
            wait_write(jnp.int32(c), c % KSLOT)

    return k(row, col, a_t, b_t, xn1d)


def _sc_scatter(row, s, t16, zrows, zrows16):
    ne = row.shape[0]
    nchunks = ne // CB
    cpw = -(-nchunks // NW)
    mesh = plsc.VectorSubcoreMesh(core_axis_name="c", subcore_axis_name="s")
    nsteps = cpw + 2
    nu = -(-nsteps // KSLOT)

    @functools.partial(
        pl.kernel,
        mesh=mesh,
        out_type=[jax.ShapeDtypeStruct((NPAD, HID), _f32),
                  jax.ShapeDtypeStruct((NPAD, HID), _f32),
                  jax.ShapeDtypeStruct((NPAD, 16), _f32),
                  jax.ShapeDtypeStruct((NPAD, 16), _f32)],
        scratch_types=([pltpu.VMEM((CB,), jnp.int32)] * KSLOT
                       + [pltpu.VMEM((CB, HID), _f32)] * KSLOT
                       + [pltpu.VMEM((CB, 16), _f32)] * KSLOT
                       + [pltpu.VMEM_SHARED((NPAD, HID), _f32)]
                       + [pltpu.VMEM_SHARED((NPAD, 16), _f32)]
                       + [pltpu.SemaphoreType.DMA] * (3 * KSLOT)),
        compiler_params=pltpu.CompilerParams(needs_layout_passes=False, use_tc_tiling_on_sc=False),
    )
    def k(row_h, s_h, t_h, z_h, z16_h, p0_h, p1_h, q0_h, q1_h, *scr):
        idx = scr[0:KSLOT]
        buf = scr[KSLOT:2 * KSLOT]
        tbuf = scr[2 * KSLOT:3 * KSLOT]
        acc = scr[3 * KSLOT]
        acc2 = scr[3 * KSLOT + 1]
        l_sem = scr[3 * KSLOT + 2:4 * KSLOT + 2]
        a_sem = scr[4 * KSLOT + 2:5 * KSLOT + 2]
        t_sem = scr[5 * KSLOT + 2:6 * KSLOT + 2]
        cid = lax.axis_index("c")
        sid = lax.axis_index("s")
        wid = sid * NC + cid
        rbase = sid * ROWS_PER_SUB

        pltpu.sync_copy(z_h, acc.at[pl.ds(rbase, ROWS_PER_SUB)])
        pltpu.sync_copy(z16_h, acc2.at[pl.ds(rbase, ROWS_PER_SUB)])
        plsc.subcore_barrier()

        def guard(t):
            return (t >= 0) & (wid + t * NW < nchunks)

        def chunk_off(t):
            return (wid + t * NW) * CB

        def issue_load(t, z):
            @pl.when(guard(t))
            def _():
                off = chunk_off(t)
                pltpu.async_copy(row_h.at[pl.ds(off, CB)], idx[z], l_sem[z])
                pltpu.async_copy(s_h.at[pl.ds(off, CB)], buf[z], l_sem[z])
                pltpu.async_copy(t_h.at[pl.ds(off, CB), pl.ds(0, 16)],
                                 tbuf[z], l_sem[z])

        def start_add(t, z):
            @pl.when(guard(t))
            def _():
                pltpu.make_async_copy(row_h.at[pl.ds(0, CB)], idx[z],
                                      l_sem[z]).wait()
                pltpu.make_async_copy(s_h.at[pl.ds(0, CB)], buf[z],
                                      l_sem[z]).wait()
                pltpu.make_async_copy(t_h.at[pl.ds(0, CB), pl.ds(0, 16)],
                                      tbuf[z], l_sem[z]).wait()
                pltpu.async_copy(buf[z], acc.at[idx[z]], a_sem[z], add=True)
                pltpu.async_copy(tbuf[z], acc2.at[idx[z]], t_sem[z],
                                 add=True)

        def wait_add(t, z):
            @pl.when(guard(t))
            def _():
                pltpu.make_async_copy(buf[z], acc.at[idx[z]],
                                      a_sem[z]).wait()
                pltpu.make_async_copy(tbuf[z], acc2.at[idx[z]],
                                      t_sem[z]).wait()

        def body(u, carry):
            for z in range(KSLOT):
                t = u * KSLOT + z
                wait_add(t - KSLOT, z)
                issue_load(t, z)
                start_add(t - 1, (z - 1) % KSLOT)
            return carry

        lax.fori_loop(0, nu, body, 0)
        for c in range(max(nu * KSLOT - KSLOT, 0), cpw):
            wait_add(jnp.int32(c), c % KSLOT)
        plsc.subcore_barrier()

        @pl.when(cid == 0)
        def _():
            pltpu.sync_copy(acc.at[pl.ds(rbase, ROWS_PER_SUB)],
                            p0_h.at[pl.ds(rbase, ROWS_PER_SUB)])
            pltpu.sync_copy(acc2.at[pl.ds(rbase, ROWS_PER_SUB)],
                            q0_h.at[pl.ds(rbase, ROWS_PER_SUB)])

        @pl.when(cid == 1)
        def _():
            pltpu.sync_copy(acc.at[pl.ds(rbase, ROWS_PER_SUB)],
                            p1_h.at[pl.ds(rbase, ROWS_PER_SUB)])
            pltpu.sync_copy(acc2.at[pl.ds(rbase, ROWS_PER_SUB)],
                            q1_h.at[pl.ds(rbase, ROWS_PER_SUB)])

    return k(row, s, t16, zrows, zrows16)


def _xnorm(x4):
    def body(x_ref, o_ref):
        xv = x_ref[...]
        xs = xv - jnp.min(xv, axis=0, keepdims=True)
        xn = xs / jnp.max(xs, axis=0, keepdims=True)
        lane = lax.broadcasted_iota(jnp.int32, xn.shape, 1)
        o_ref[...] = jnp.where(lane < 3, xn, 0.0)

    return pl.pallas_call(
        body, out_shape=jax.ShapeDtypeStruct((N, 4), _f32))(x4)


def _emb(h0, w, b):
    def body(h_ref, w_ref, b_ref, o_ref):
        o_ref[...] = jnp.dot(h_ref[...], w_ref[...],
                             preferred_element_type=_f32,
                             precision=lax.Precision.HIGHEST) + b_ref[...]

    return pl.pallas_call(
        body,
        grid=(N // NBLK,),
        in_specs=[pl.BlockSpec((NBLK, HID), lambda i: (i, 0)),
                  pl.BlockSpec((HID, HID), lambda i: (0, 0)),
                  pl.BlockSpec((1, HID), lambda i: (0, 0))],
        out_specs=pl.BlockSpec((NBLK, HID), lambda i: (i, 0)),
        out_shape=jax.ShapeDtypeStruct((N, HID), _f32),
    )(h0, w, b)


def _prep(h, w1a, w1b, b1):
    def body(h_ref, wa_ref, wb_ref, b_ref, a_ref, bx_ref):
        hb = h_ref[...]
        a_ref[...] = jnp.dot(hb, wa_ref[...],
                             preferred_element_type=_f32,
                             precision=lax.Precision.HIGHEST) + b_ref[...]
        bx_ref[...] = jnp.dot(hb, wb_ref[...], preferred_element_type=_f32,
                             precision=lax.Precision.HIGHEST)

    return pl.pallas_call(
        body,
        grid=(N // NBLK,),
        in_specs=[pl.BlockSpec((NBLK, HID), lambda i: (i, 0)),
                  pl.BlockSpec((HID, HID), lambda i: (0, 0)),
                  pl.BlockSpec((HID, HID), lambda i: (0, 0)),
                  pl.BlockSpec((1, HID), lambda i: (0, 0))],
        out_specs=[pl.BlockSpec((NBLK, HID), lambda i: (i, 0)),
                   pl.BlockSpec((NBLK, HID), lambda i: (i, 0))],
        out_shape=[jax.ShapeDtypeStruct((N, HID), _f32),
                   jax.ShapeDtypeStruct((N, HID), _f32)],
    )(h, w1a, w1b, b1)


def _edge(ga, gb, r16, ea, w1e, wr, w2, b2, wc1, bc1, wc2):
    def body(ga_ref, gb_ref, r_ref, ea_ref, w1e_ref, wr_ref, w2_ref, b2_ref,
             wc1_ref, bc1_ref, wc2_ref, s_ref, cm_ref):
        radial = r_ref[...][:, :1]
        pre = (ga_ref[...] + gb_ref[...] + radial * wr_ref[...]
               + jnp.dot(ea_ref[...], w1e_ref[...],
                         preferred_element_type=_f32,
                             precision=lax.Precision.HIGHEST))
        u = _silu(pre)
        ef = _silu(jnp.dot(u, w2_ref[...], preferred_element_type=_f32,
                             precision=lax.Precision.HIGHEST)
                   + b2_ref[...])
        v = _silu(jnp.dot(ef, wc1_ref[...], preferred_element_type=_f32,
                             precision=lax.Precision.HIGHEST)
                  + bc1_ref[...])
        cm = jnp.dot(v, wc2_ref[...], preferred_element_type=_f32,
                             precision=lax.Precision.HIGHEST)
        s_ref[...] = ef
        trans = r_ref[...][:, 1:4] * cm[:, :1]
        cm_ref[...] = jnp.concatenate(
            [trans, jnp.zeros((trans.shape[0], HID - 3), _f32)], axis=1)

    ne = ga.shape[0]
    return pl.pallas_call(
        body,
        grid=(ne // EBLK,),
        in_specs=[pl.BlockSpec((EBLK, HID), lambda i: (i, 0)),
                  pl.BlockSpec((EBLK, HID), lambda i: (i, 0)),
                  pl.BlockSpec((EBLK, HID), lambda i: (i, 0)),
                  pl.BlockSpec((EBLK, 16), lambda i: (i, 0)),
                  pl.BlockSpec((16, HID), lambda i: (0, 0)),
                  pl.BlockSpec((1, HID), lambda i: (0, 0)),
                  pl.BlockSpec((HID, HID), lambda i: (0, 0)),
                  pl.BlockSpec((1, HID), lambda i: (0, 0)),
                  pl.BlockSpec((HID, HID), lambda i: (0, 0)),
                  pl.BlockSpec((1, HID), lambda i: (0, 0)),
                  pl.BlockSpec((HID, 8), lambda i: (0, 0))],
        out_specs=[pl.BlockSpec((EBLK, HID), lambda i: (i, 0)),
                   pl.BlockSpec((EBLK, HID), lambda i: (i, 0))],
        out_shape=[jax.ShapeDtypeStruct((ne, HID), _f32),
                   jax.ShapeDtypeStruct((ne, HID), _f32)],
    )(ga, gb, r16, ea, w1e, wr, w2, b2, wc1, bc1, wc2)


def _node(h, h0, ps, qs, xn, xw, wn1, bn1, wn2, bn2):
    nps = len(ps)

    def body(h_ref, h0_ref, *rest):
        p_refs = rest[0:nps]
        q_refs = rest[nps:2 * nps]
        (xn_ref, xw_ref, wn1_ref, bn1_ref, wn2_ref, bn2_ref,
         hn_ref, xo_ref) = rest[2 * nps:]
        hb = h_ref[...]
        nagg = p_refs[0][...]
        for pr in p_refs[1:]:
            nagg = nagg + pr[...]
        agg = q_refs[0][...][:, :4]
        for qr in q_refs[1:]:
            agg = agg + qr[...][:, :4]
        w1 = wn1_ref[...]
        t = (jnp.dot(hb, w1[0:HID], preferred_element_type=_f32,
                             precision=lax.Precision.HIGHEST)
             + jnp.dot(nagg, w1[HID:2 * HID], preferred_element_type=_f32,
                             precision=lax.Precision.HIGHEST)
             + jnp.dot(h0_ref[...], w1[2 * HID:3 * HID],
                       preferred_element_type=_f32,
                             precision=lax.Precision.HIGHEST)
             + bn1_ref[...])
        hn_ref[...] = hb + jnp.dot(_silu(t), wn2_ref[...],
                                   preferred_element_type=_f32,
                             precision=lax.Precision.HIGHEST) + bn2_ref[...]
        xo_ref[...] = xn_ref[...] + agg / xw_ref[...]

    return pl.pallas_call(
        body,
        grid=(N // NBLK,),
        in_specs=([pl.BlockSpec((NBLK, HID), lambda i: (i, 0))] * 2
                  + [pl.BlockSpec((NBLK, HID), lambda i: (i, 0))] * nps
                  + [pl.BlockSpec((NBLK, 16), lambda i: (i, 0))] * nps
                  + [pl.BlockSpec((NBLK, 4), lambda i: (i, 0)),
                     pl.BlockSpec((NBLK, 1), lambda i: (i, 0)),
                     pl.BlockSpec((3 * HID, HID), lambda i: (0, 0)),
                     pl.BlockSpec((1, HID), lambda i: (0, 0)),
                     pl.BlockSpec((HID, HID), lambda i: (0, 0)),
                     pl.BlockSpec((1, HID), lambda i: (0, 0))]),
        out_specs=[pl.BlockSpec((NBLK, HID), lambda i: (i, 0)),
                   pl.BlockSpec((NBLK, 4), lambda i: (i, 0))],
        out_shape=[jax.ShapeDtypeStruct((N, HID), _f32),
                   jax.ShapeDtypeStruct((N, 4), _f32)],
    )(h, h0, *ps, *qs, xn, xw, wn1, bn1, wn2, bn2)


def _dec(h, wd1, bd1, wd2, bd2, wg1, bg1, wg2, bg2):
    def body(h_ref, wd1_ref, bd1_ref, wd2_ref, bd2_ref,
             wg1_ref, bg1_ref, wg2_ref, bg2_ref, o_ref):
        t = _silu(jnp.dot(h_ref[...], wd1_ref[...],
                          preferred_element_type=_f32,
                             precision=lax.Precision.HIGHEST) + bd1_ref[...])
        t = jnp.dot(t, wd2_ref[...], preferred_element_type=_f32,
                             precision=lax.Precision.HIGHEST) + bd2_ref[...]
        u = _silu(jnp.dot(t, wg1_ref[...],
                          preferred_element_type=_f32,
                             precision=lax.Precision.HIGHEST) + bg1_ref[...])
        o_ref[...] = jnp.dot(u, wg2_ref[...],
                             preferred_element_type=_f32,
                             precision=lax.Precision.HIGHEST) + bg2_ref[...]

    return pl.pallas_call(
        body,
        grid=(N // NBLK,),
        in_specs=[pl.BlockSpec((NBLK, HID), lambda i: (i, 0)),
                  pl.BlockSpec((HID, HID), lambda i: (0, 0)),
                  pl.BlockSpec((1, HID), lambda i: (0, 0)),
                  pl.BlockSpec((HID, HID), lambda i: (0, 0)),
                  pl.BlockSpec((1, HID), lambda i: (0, 0)),
                  pl.BlockSpec((HID, HID), lambda i: (0, 0)),
                  pl.BlockSpec((1, HID), lambda i: (0, 0)),
                  pl.BlockSpec((HID, 21), lambda i: (0, 0)),
                  pl.BlockSpec((1, 21), lambda i: (0, 0))],
        out_specs=pl.BlockSpec((NBLK, 21), lambda i: (i, 0)),
        out_shape=jax.ShapeDtypeStruct((N, 21), _f32),
    )(h, wd1, bd1, wd2, bd2, wg1, bg1, wg2, bg2)


def kernel(h0, x, edges, edge_attr, x_weights, params):
    row = edges[0]
    col = edges[1]
    x4 = jnp.concatenate([x, jnp.zeros((N, 1), _f32)], axis=1)
    zrows = jnp.zeros((ROWS_PER_SUB, HID), _f32)
    zrows16 = jnp.zeros((ROWS_PER_SUB, 16), _f32)

    wemb, bemb = params['emb']
    h = _emb(h0, wemb, bemb.reshape(1, HID))

    for lp in params['layers']:
        W1, b1 = lp['edge_mlp'][0]
        W2, b2 = lp['edge_mlp'][1]
        Wc1, bc1 = lp['coord_mlp'][0]
        Wc2, _ = lp['coord_mlp'][1]
        Wn1, bn1 = lp['node_mlp'][0]
        Wn2, bn2 = lp['node_mlp'][1]
        w1a = W1[0:HID]
        w1b = W1[HID:2 * HID]
        wr = W1[2 * HID:2 * HID + 1]
        w1e = W1[2 * HID + 1:]
        wc2p = jnp.pad(Wc2, ((0, 0), (0, 7)))

        xn = _xnorm(x4)
        xn1d = jnp.ravel(xn[:, :3])
        a_t, b_t = _prep(h, w1a, w1b, b1.reshape(1, HID))
        ps, qs = [], []
        for hh in range(NHALF):
            lo = hh * EH
            rowh = lax.dynamic_slice_in_dim(row, lo, EH)
            colh = lax.dynamic_slice_in_dim(col, lo, EH)
            eah = lax.dynamic_slice_in_dim(edge_attr, lo, EH)
            ga, gb, r16 = _sc_gather(rowh, colh, a_t, b_t, xn1d)
            sh, t16 = _edge(ga, gb, r16, eah, w1e, wr, W2,
                            b2.reshape(1, HID), Wc1, bc1.reshape(1, HID),
                            wc2p)
            p0, p1, q0, q1 = _sc_scatter(rowh, sh, t16, zrows, zrows16)
            ps += [p0, p1]
            qs += [q0, q1]
        h, x4 = _node(h, h0, ps, qs, xn, x_weights,
                      Wn1, bn1.reshape(1, HID), Wn2, bn2.reshape(1, HID))

    Wd1, bd1 = params['node_dec'][0]
    Wd2, bd2 = params['node_dec'][1]
    Wg1, bg1 = params['graph_dec'][0]
    Wg2, bg2 = params['graph_dec'][1]
    return _dec(h, Wd1, bd1.reshape(1, HID), Wd2, bd2.reshape(1, HID),
                Wg1, bg1.reshape(1, HID), Wg2, bg2.reshape(1, 21))


# speedup vs baseline: 1.3307x; 1.3307x over previous
import functools

import jax
import jax.numpy as jnp
from jax import lax
from jax.experimental import pallas as pl
from jax.experimental.pallas import tpu as pltpu
from jax.experimental.pallas import tpu_sc as plsc

N = 10000
E = 320000
HID = 128
NPAD = 10240
NC = 2
NS = 16
NW = NC * NS
CB = 64
NHALF = 2
EH = E // NHALF
KSLOT = 4
ROWS_PER_SUB = NPAD // NS
EBLK = 3200
NBLK = 1000

_f32 = jnp.float32


def _silu(v):
    return v * jax.nn.sigmoid(v)


def _iota16():
    return lax.iota(jnp.int32, 16)


def _sc_gather(row, col, a_t, b_t, xn1d):
    ne = row.shape[0]
    nchunks = ne // CB
    cpw = -(-nchunks // NW)
    mesh = plsc.VectorSubcoreMesh(core_axis_name="c", subcore_axis_name="s")
    nsteps = cpw + 2
    nu = -(-nsteps // KSLOT)

    @functools.partial(
        pl.kernel,
        mesh=mesh,
        out_type=[jax.ShapeDtypeStruct((ne, HID), _f32),
                  jax.ShapeDtypeStruct((ne, HID), _f32),
                  jax.ShapeDtypeStruct((ne, HID), _f32)],
        scratch_types=([pltpu.VMEM((CB,), jnp.int32)] * (2 * KSLOT)
                       + [pltpu.VMEM((CB, HID), _f32)] * (2 * KSLOT)
                       + [pltpu.VMEM((CB, 16), _f32)] * KSLOT
                       + [pltpu.VMEM((3 * N,), _f32)]
                       + [pltpu.SemaphoreType.DMA] * (3 * KSLOT + 1)),
        compiler_params=pltpu.CompilerParams(needs_layout_passes=False, use_tc_tiling_on_sc=False),
    )
    def k(row_h, col_h, a_h, b_h, xn_h, ga_h, gb_h, r16_h, *scr):
        idxa = scr[0:KSLOT]
        idxb = scr[KSLOT:2 * KSLOT]
        bufa = scr[2 * KSLOT:3 * KSLOT]
        bufb = scr[3 * KSLOT:4 * KSLOT]
        rbuf = scr[4 * KSLOT:5 * KSLOT]
        xn_v = scr[5 * KSLOT]
        i_sem = scr[5 * KSLOT + 1:6 * KSLOT + 1]
        g_sem = scr[6 * KSLOT + 1:7 * KSLOT + 1]
        w_sem = scr[7 * KSLOT + 1:8 * KSLOT + 1]
        x_sem = scr[8 * KSLOT + 1]
        wid = lax.axis_index("s") * NC + lax.axis_index("c")

        pltpu.async_copy(xn_h, xn_v, x_sem).wait()

        def guard(t):
            return (t >= 0) & (wid + t * NW < nchunks)

        def chunk_off(t):
            return (wid + t * NW) * CB

        def issue_idx(t, z):
            @pl.when(guard(t))
            def _():
                off = chunk_off(t)
                pltpu.async_copy(row_h.at[pl.ds(off, CB)], idxa[z], i_sem[z])
                pltpu.async_copy(col_h.at[pl.ds(off, CB)], idxb[z], i_sem[z])

        def start_gather(t, z):
            @pl.when(guard(t))
            def _():
                pltpu.make_async_copy(row_h.at[pl.ds(0, CB)], idxa[z],
                                      i_sem[z]).wait()
                pltpu.make_async_copy(col_h.at[pl.ds(0, CB)], idxb[z],
                                      i_sem[z]).wait()
                pltpu.async_copy(a_h.at[idxa[z]], bufa[z], g_sem[z])
                pltpu.async_copy(b_h.at[idxb[z]], bufb[z], g_sem[z])
                i16 = _iota16()
                lane0 = jnp.zeros((16,), jnp.int32)
                for g in range(CB // 16):
                    rv = idxa[z][pl.ds(g * 16, 16)]
                    cv = idxb[z][pl.ds(g * 16, 16)]
                    r4 = rv * 3
                    c4 = cv * 3
                    rad = None
                    for comp in range(3):
                        xr = plsc.load_gather(xn_v, [r4 + comp])
                        xc = plsc.load_gather(xn_v, [c4 + comp])
                        dx = xr - xc
                        plsc.store_scatter(rbuf[z], [g * 16 + i16,
                                                     lane0 + 1 + comp], dx)
                        rad = dx * dx if rad is None else rad + dx * dx
                    plsc.store_scatter(rbuf[z], [g * 16 + i16, lane0], rad)

        def start_write(t, z):
            @pl.when(guard(t))
            def _():
                off = chunk_off(t)
                pltpu.make_async_copy(a_h.at[idxa[z]], bufa[z],
                                      g_sem[z]).wait()
                pltpu.make_async_copy(b_h.at[idxb[z]], bufb[z],
                                      g_sem[z]).wait()
                pltpu.async_copy(bufa[z], ga_h.at[pl.ds(off, CB)], w_sem[z])
                pltpu.async_copy(bufb[z], gb_h.at[pl.ds(off, CB)], w_sem[z])
                pltpu.async_copy(rbuf[z],
                                 r16_h.at[pl.ds(off, CB), pl.ds(0, 16)],
                                 w_sem[z])

        def wait_write(t, z):
            @pl.when(guard(t))
            def _():
                off = chunk_off(t)
                pltpu.make_async_copy(bufa[z], ga_h.at[pl.ds(off, CB)],
                                      w_sem[z]).wait()
                pltpu.make_async_copy(bufb[z], gb_h.at[pl.ds(off, CB)],
                                      w_sem[z]).wait()
                pltpu.make_async_copy(rbuf[z],
                                      r16_h.at[pl.ds(off, CB),
                                               pl.ds(0, 16)],
                                      w_sem[z]).wait()

        def body(u, carry):
            for z in range(KSLOT):
                t = u * KSLOT + z
                wait_write(t - KSLOT, z)
                issue_idx(t, z)
                start_gather(t - 1, (z - 1) % KSLOT)
                start_write(t - 2, (z - 2) % KSLOT)
            return carry

        lax.fori_loop(0, nu, body, 0)
        for c in range(max(nu * KSLOT - KSLOT, 0), cpw):
            wait_write(jnp.int32(c), c % KSLOT)

    return k(row, col, a_t, b_t, xn1d)


def _sc_scatter(row, s, t16, zrows, zrows16):
    ne = row.shape[0]
    nchunks = ne // CB
    cpw = -(-nchunks // NW)
    mesh = plsc.VectorSubcoreMesh(core_axis_name="c", subcore_axis_name="s")
    nsteps = cpw + 2
    nu = -(-nsteps // KSLOT)

    @functools.partial(
        pl.kernel,
        mesh=mesh,
        out_type=[jax.ShapeDtypeStruct((NPAD, HID), _f32),
                  jax.ShapeDtypeStruct((NPAD, HID), _f32),
                  jax.ShapeDtypeStruct((NPAD, 16), _f32),
                  jax.ShapeDtypeStruct((NPAD, 16), _f32)],
        scratch_types=([pltpu.VMEM((CB,), jnp.int32)] * KSLOT
                       + [pltpu.VMEM((CB, HID), _f32)] * KSLOT
                       + [pltpu.VMEM((CB, 16), _f32)] * KSLOT
                       + [pltpu.VMEM_SHARED((NPAD, HID), _f32)]
                       + [pltpu.VMEM_SHARED((NPAD, 16), _f32)]
                       + [pltpu.SemaphoreType.DMA] * (3 * KSLOT)),
        compiler_params=pltpu.CompilerParams(needs_layout_passes=False, use_tc_tiling_on_sc=False),
    )
    def k(row_h, s_h, t_h, z_h, z16_h, p0_h, p1_h, q0_h, q1_h, *scr):
        idx = scr[0:KSLOT]
        buf = scr[KSLOT:2 * KSLOT]
        tbuf = scr[2 * KSLOT:3 * KSLOT]
        acc = scr[3 * KSLOT]
        acc2 = scr[3 * KSLOT + 1]
        l_sem = scr[3 * KSLOT + 2:4 * KSLOT + 2]
        a_sem = scr[4 * KSLOT + 2:5 * KSLOT + 2]
        t_sem = scr[5 * KSLOT + 2:6 * KSLOT + 2]
        cid = lax.axis_index("c")
        sid = lax.axis_index("s")
        wid = sid * NC + cid
        rbase = sid * ROWS_PER_SUB

        pltpu.sync_copy(z_h, acc.at[pl.ds(rbase, ROWS_PER_SUB)])
        pltpu.sync_copy(z16_h, acc2.at[pl.ds(rbase, ROWS_PER_SUB)])
        plsc.subcore_barrier()

        def guard(t):
            return (t >= 0) & (wid + t * NW < nchunks)

        def chunk_off(t):
            return (wid + t * NW) * CB

        def issue_load(t, z):
            @pl.when(guard(t))
            def _():
                off = chunk_off(t)
                pltpu.async_copy(row_h.at[pl.ds(off, CB)], idx[z], l_sem[z])
                pltpu.async_copy(s_h.at[pl.ds(off, CB)], buf[z], l_sem[z])
                pltpu.async_copy(t_h.at[pl.ds(off, CB), pl.ds(0, 16)],
                                 tbuf[z], l_sem[z])

        def start_add(t, z):
            @pl.when(guard(t))
            def _():
                pltpu.make_async_copy(row_h.at[pl.ds(0, CB)], idx[z],
                                      l_sem[z]).wait()
                pltpu.make_async_copy(s_h.at[pl.ds(0, CB)], buf[z],
                                      l_sem[z]).wait()
                pltpu.make_async_copy(t_h.at[pl.ds(0, CB), pl.ds(0, 16)],
                                      tbuf[z], l_sem[z]).wait()
                pltpu.async_copy(buf[z], acc.at[idx[z]], a_sem[z], add=True)
                pltpu.async_copy(tbuf[z], acc2.at[idx[z]], t_sem[z],
                                 add=True)

        def wait_add(t, z):
            @pl.when(guard(t))
            def _():
                pltpu.make_async_copy(buf[z], acc.at[idx[z]],
                                      a_sem[z]).wait()
                pltpu.make_async_copy(tbuf[z], acc2.at[idx[z]],
                                      t_sem[z]).wait()

        def body(u, carry):
            for z in range(KSLOT):
                t = u * KSLOT + z
                wait_add(t - KSLOT, z)
                issue_load(t, z)
                start_add(t - 1, (z - 1) % KSLOT)
            return carry

        lax.fori_loop(0, nu, body, 0)
        for c in range(max(nu * KSLOT - KSLOT, 0), cpw):
            wait_add(jnp.int32(c), c % KSLOT)
        plsc.subcore_barrier()

        @pl.when(cid == 0)
        def _():
            pltpu.sync_copy(acc.at[pl.ds(rbase, ROWS_PER_SUB)],
                            p0_h.at[pl.ds(rbase, ROWS_PER_SUB)])
            pltpu.sync_copy(acc2.at[pl.ds(rbase, ROWS_PER_SUB)],
                            q0_h.at[pl.ds(rbase, ROWS_PER_SUB)])

        @pl.when(cid == 1)
        def _():
            pltpu.sync_copy(acc.at[pl.ds(rbase, ROWS_PER_SUB)],
                            p1_h.at[pl.ds(rbase, ROWS_PER_SUB)])
            pltpu.sync_copy(acc2.at[pl.ds(rbase, ROWS_PER_SUB)],
                            q1_h.at[pl.ds(rbase, ROWS_PER_SUB)])

    return k(row, s, t16, zrows, zrows16)


def _xnorm(x4):
    def body(x_ref, o_ref):
        xv = x_ref[...]
        xs = xv - jnp.min(xv, axis=0, keepdims=True)
        xn = xs / jnp.max(xs, axis=0, keepdims=True)
        lane = lax.broadcasted_iota(jnp.int32, xn.shape, 1)
        o_ref[...] = jnp.where(lane < 3, xn, 0.0)

    return pl.pallas_call(
        body, out_shape=jax.ShapeDtypeStruct((N, 4), _f32))(x4)


def _emb(h0, w, b):
    def body(h_ref, w_ref, b_ref, o_ref):
        o_ref[...] = jnp.dot(h_ref[...], w_ref[...],
                             preferred_element_type=_f32,
                             precision=lax.Precision.HIGHEST) + b_ref[...]

    return pl.pallas_call(
        body,
        grid=(N // NBLK,),
        in_specs=[pl.BlockSpec((NBLK, HID), lambda i: (i, 0)),
                  pl.BlockSpec((HID, HID), lambda i: (0, 0)),
                  pl.BlockSpec((1, HID), lambda i: (0, 0))],
        out_specs=pl.BlockSpec((NBLK, HID), lambda i: (i, 0)),
        out_shape=jax.ShapeDtypeStruct((N, HID), _f32),
    )(h0, w, b)


def _prep(h, w1a, w1b, b1):
    def body(h_ref, wa_ref, wb_ref, b_ref, a_ref, bx_ref):
        hb = h_ref[...]
        a_ref[...] = jnp.dot(hb, wa_ref[...],
                             preferred_element_type=_f32,
                             precision=lax.Precision.HIGHEST) + b_ref[...]
        bx_ref[...] = jnp.dot(hb, wb_ref[...], preferred_element_type=_f32,
                             precision=lax.Precision.HIGHEST)

    return pl.pallas_call(
        body,
        grid=(N // NBLK,),
        in_specs=[pl.BlockSpec((NBLK, HID), lambda i: (i, 0)),
                  pl.BlockSpec((HID, HID), lambda i: (0, 0)),
                  pl.BlockSpec((HID, HID), lambda i: (0, 0)),
                  pl.BlockSpec((1, HID), lambda i: (0, 0))],
        out_specs=[pl.BlockSpec((NBLK, HID), lambda i: (i, 0)),
                   pl.BlockSpec((NBLK, HID), lambda i: (i, 0))],
        out_shape=[jax.ShapeDtypeStruct((N, HID), _f32),
                   jax.ShapeDtypeStruct((N, HID), _f32)],
    )(h, w1a, w1b, b1)


def _edge(ga, gb, r16, ea, w1e, wr, w2, b2, wc1, bc1, wc2):
    def body(ga_ref, gb_ref, r_ref, ea_ref, w1e_ref, wr_ref, w2_ref, b2_ref,
             wc1_ref, bc1_ref, wc2_ref, s_ref, cm_ref):
        radial = r_ref[...][:, :1]
        pre = (ga_ref[...] + gb_ref[...] + radial * wr_ref[...]
               + jnp.dot(ea_ref[...], w1e_ref[...],
                         preferred_element_type=_f32,
                             precision=lax.Precision.HIGHEST))
        u = _silu(pre)
        ef = _silu(jnp.dot(u, w2_ref[...], preferred_element_type=_f32)
                   + b2_ref[...])
        v = _silu(jnp.dot(ef, wc1_ref[...], preferred_element_type=_f32)
                  + bc1_ref[...])
        cm = jnp.dot(v, wc2_ref[...], preferred_element_type=_f32,
                             precision=lax.Precision.HIGHEST)
        s_ref[...] = ef
        trans = r_ref[...][:, 1:4] * cm[:, :1]
        cm_ref[...] = jnp.concatenate(
            [trans, jnp.zeros((trans.shape[0], HID - 3), _f32)], axis=1)

    ne = ga.shape[0]
    return pl.pallas_call(
        body,
        grid=(ne // EBLK,),
        in_specs=[pl.BlockSpec((EBLK, HID), lambda i: (i, 0)),
                  pl.BlockSpec((EBLK, HID), lambda i: (i, 0)),
                  pl.BlockSpec((EBLK, HID), lambda i: (i, 0)),
                  pl.BlockSpec((EBLK, 16), lambda i: (i, 0)),
                  pl.BlockSpec((16, HID), lambda i: (0, 0)),
                  pl.BlockSpec((1, HID), lambda i: (0, 0)),
                  pl.BlockSpec((HID, HID), lambda i: (0, 0)),
                  pl.BlockSpec((1, HID), lambda i: (0, 0)),
                  pl.BlockSpec((HID, HID), lambda i: (0, 0)),
                  pl.BlockSpec((1, HID), lambda i: (0, 0)),
                  pl.BlockSpec((HID, 8), lambda i: (0, 0))],
        out_specs=[pl.BlockSpec((EBLK, HID), lambda i: (i, 0)),
                   pl.BlockSpec((EBLK, HID), lambda i: (i, 0))],
        out_shape=[jax.ShapeDtypeStruct((ne, HID), _f32),
                   jax.ShapeDtypeStruct((ne, HID), _f32)],
    )(ga, gb, r16, ea, w1e, wr, w2, b2, wc1, bc1, wc2)


def _node(h, h0, ps, qs, xn, xw, wn1, bn1, wn2, bn2):
    nps = len(ps)

    def body(h_ref, h0_ref, *rest):
        p_refs = rest[0:nps]
        q_refs = rest[nps:2 * nps]
        (xn_ref, xw_ref, wn1_ref, bn1_ref, wn2_ref, bn2_ref,
         hn_ref, xo_ref) = rest[2 * nps:]
        hb = h_ref[...]
        nagg = p_refs[0][...]
        for pr in p_refs[1:]:
            nagg = nagg + pr[...]
        agg = q_refs[0][...][:, :4]
        for qr in q_refs[1:]:
            agg = agg + qr[...][:, :4]
        w1 = wn1_ref[...]
        t = (jnp.dot(hb, w1[0:HID], preferred_element_type=_f32,
                             precision=lax.Precision.HIGHEST)
             + jnp.dot(nagg, w1[HID:2 * HID], preferred_element_type=_f32,
                             precision=lax.Precision.HIGHEST)
             + jnp.dot(h0_ref[...], w1[2 * HID:3 * HID],
                       preferred_element_type=_f32,
                             precision=lax.Precision.HIGHEST)
             + bn1_ref[...])
        hn_ref[...] = hb + jnp.dot(_silu(t), wn2_ref[...],
                                   preferred_element_type=_f32,
                             precision=lax.Precision.HIGHEST) + bn2_ref[...]
        xo_ref[...] = xn_ref[...] + agg / xw_ref[...]

    return pl.pallas_call(
        body,
        grid=(N // NBLK,),
        in_specs=([pl.BlockSpec((NBLK, HID), lambda i: (i, 0))] * 2
                  + [pl.BlockSpec((NBLK, HID), lambda i: (i, 0))] * nps
                  + [pl.BlockSpec((NBLK, 16), lambda i: (i, 0))] * nps
                  + [pl.BlockSpec((NBLK, 4), lambda i: (i, 0)),
                     pl.BlockSpec((NBLK, 1), lambda i: (i, 0)),
                     pl.BlockSpec((3 * HID, HID), lambda i: (0, 0)),
                     pl.BlockSpec((1, HID), lambda i: (0, 0)),
                     pl.BlockSpec((HID, HID), lambda i: (0, 0)),
                     pl.BlockSpec((1, HID), lambda i: (0, 0))]),
        out_specs=[pl.BlockSpec((NBLK, HID), lambda i: (i, 0)),
                   pl.BlockSpec((NBLK, 4), lambda i: (i, 0))],
        out_shape=[jax.ShapeDtypeStruct((N, HID), _f32),
                   jax.ShapeDtypeStruct((N, 4), _f32)],
    )(h, h0, *ps, *qs, xn, xw, wn1, bn1, wn2, bn2)


def _dec(h, wd1, bd1, wd2, bd2, wg1, bg1, wg2, bg2):
    def body(h_ref, wd1_ref, bd1_ref, wd2_ref, bd2_ref,
             wg1_ref, bg1_ref, wg2_ref, bg2_ref, o_ref):
        t = _silu(jnp.dot(h_ref[...], wd1_ref[...],
                          preferred_element_type=_f32,
                             precision=lax.Precision.HIGHEST) + bd1_ref[...])
        t = jnp.dot(t, wd2_ref[...], preferred_element_type=_f32,
                             precision=lax.Precision.HIGHEST) + bd2_ref[...]
        u = _silu(jnp.dot(t, wg1_ref[...],
                          preferred_element_type=_f32,
                             precision=lax.Precision.HIGHEST) + bg1_ref[...])
        o_ref[...] = jnp.dot(u, wg2_ref[...],
                             preferred_element_type=_f32,
                             precision=lax.Precision.HIGHEST) + bg2_ref[...]

    return pl.pallas_call(
        body,
        grid=(N // NBLK,),
        in_specs=[pl.BlockSpec((NBLK, HID), lambda i: (i, 0)),
                  pl.BlockSpec((HID, HID), lambda i: (0, 0)),
                  pl.BlockSpec((1, HID), lambda i: (0, 0)),
                  pl.BlockSpec((HID, HID), lambda i: (0, 0)),
                  pl.BlockSpec((1, HID), lambda i: (0, 0)),
                  pl.BlockSpec((HID, HID), lambda i: (0, 0)),
                  pl.BlockSpec((1, HID), lambda i: (0, 0)),
                  pl.BlockSpec((HID, 21), lambda i: (0, 0)),
                  pl.BlockSpec((1, 21), lambda i: (0, 0))],
        out_specs=pl.BlockSpec((NBLK, 21), lambda i: (i, 0)),
        out_shape=jax.ShapeDtypeStruct((N, 21), _f32),
    )(h, wd1, bd1, wd2, bd2, wg1, bg1, wg2, bg2)


def kernel(h0, x, edges, edge_attr, x_weights, params):
    row = edges[0]
    col = edges[1]
    x4 = jnp.concatenate([x, jnp.zeros((N, 1), _f32)], axis=1)
    zrows = jnp.zeros((ROWS_PER_SUB, HID), _f32)
    zrows16 = jnp.zeros((ROWS_PER_SUB, 16), _f32)

    wemb, bemb = params['emb']
    h = _emb(h0, wemb, bemb.reshape(1, HID))

    for lp in params['layers']:
        W1, b1 = lp['edge_mlp'][0]
        W2, b2 = lp['edge_mlp'][1]
        Wc1, bc1 = lp['coord_mlp'][0]
        Wc2, _ = lp['coord_mlp'][1]
        Wn1, bn1 = lp['node_mlp'][0]
        Wn2, bn2 = lp['node_mlp'][1]
        w1a = W1[0:HID]
        w1b = W1[HID:2 * HID]
        wr = W1[2 * HID:2 * HID + 1]
        w1e = W1[2 * HID + 1:]
        wc2p = jnp.pad(Wc2, ((0, 0), (0, 7)))

        xn = _xnorm(x4)
        xn1d = jnp.ravel(xn[:, :3])
        a_t, b_t = _prep(h, w1a, w1b, b1.reshape(1, HID))
        ps, qs = [], []
        for hh in range(NHALF):
            lo = hh * EH
            rowh = lax.dynamic_slice_in_dim(row, lo, EH)
            colh = lax.dynamic_slice_in_dim(col, lo, EH)
            eah = lax.dynamic_slice_in_dim(edge_attr, lo, EH)
            ga, gb, r16 = _sc_gather(rowh, colh, a_t, b_t, xn1d)
            sh, t16 = _edge(ga, gb, r16, eah, w1e, wr, W2,
                            b2.reshape(1, HID), Wc1, bc1.reshape(1, HID),
                            wc2p)
            p0, p1, q0, q1 = _sc_scatter(rowh, sh, t16, zrows, zrows16)
            ps += [p0, p1]
            qs += [q0, q1]
        h, x4 = _node(h, h0, ps, qs, xn, x_weights,
                      Wn1, bn1.reshape(1, HID), Wn2, bn2.reshape(1, HID))

    Wd1, bd1 = params['node_dec'][0]
    Wd2, bd2 = params['node_dec'][1]
    Wg1, bg1 = params['graph_dec'][0]
    Wg2, bg2 = params['graph_dec'][1]
    return _dec(h, Wd1, bd1.reshape(1, HID), Wd2, bd2.reshape(1, HID),
                Wg1, bg1.reshape(1, HID), Wg2, bg2.reshape(1, 21))


# speedup vs baseline: 1.8266x; 1.3726x over previous
import functools

import jax
import jax.numpy as jnp
from jax import lax
from jax.experimental import pallas as pl
from jax.experimental.pallas import tpu as pltpu
from jax.experimental.pallas import tpu_sc as plsc

N = 10000
E = 320000
HID = 128
NPAD = 10240
NC = 2
NS = 16
NW = NC * NS
CB = 64
NHALF = 2
EH = E // NHALF
KSLOT = 4
ROWS_PER_SUB = NPAD // NS
EBLK = 3200
NBLK = 1000

_f32 = jnp.float32


def _silu(v):
    return v * (0.5 * jnp.tanh(0.5 * v) + 0.5)


def _iota16():
    return lax.iota(jnp.int32, 16)


def _sc_gather(row, col, a_t, b_t, xn1d):
    ne = row.shape[0]
    nchunks = ne // CB
    cpw = -(-nchunks // NW)
    mesh = plsc.VectorSubcoreMesh(core_axis_name="c", subcore_axis_name="s")
    nsteps = cpw + 2
    nu = -(-nsteps // KSLOT)

    @functools.partial(
        pl.kernel,
        mesh=mesh,
        out_type=[jax.ShapeDtypeStruct((ne, HID), _f32),
                  jax.ShapeDtypeStruct((ne, HID), _f32),
                  jax.ShapeDtypeStruct((ne, HID), _f32)],
        scratch_types=([pltpu.VMEM((CB,), jnp.int32)] * (2 * KSLOT)
                       + [pltpu.VMEM((CB, HID), _f32)] * (2 * KSLOT)
                       + [pltpu.VMEM((CB, 16), _f32)] * KSLOT
                       + [pltpu.VMEM((3 * N,), _f32)]
                       + [pltpu.SemaphoreType.DMA] * (3 * KSLOT + 1)),
        compiler_params=pltpu.CompilerParams(needs_layout_passes=False, use_tc_tiling_on_sc=False),
    )
    def k(row_h, col_h, a_h, b_h, xn_h, ga_h, gb_h, r16_h, *scr):
        idxa = scr[0:KSLOT]
        idxb = scr[KSLOT:2 * KSLOT]
        bufa = scr[2 * KSLOT:3 * KSLOT]
        bufb = scr[3 * KSLOT:4 * KSLOT]
        rbuf = scr[4 * KSLOT:5 * KSLOT]
        xn_v = scr[5 * KSLOT]
        i_sem = scr[5 * KSLOT + 1:6 * KSLOT + 1]
        g_sem = scr[6 * KSLOT + 1:7 * KSLOT + 1]
        w_sem = scr[7 * KSLOT + 1:8 * KSLOT + 1]
        x_sem = scr[8 * KSLOT + 1]
        wid = lax.axis_index("s") * NC + lax.axis_index("c")

        pltpu.async_copy(xn_h, xn_v, x_sem).wait()

        def guard(t):
            return (t >= 0) & (wid + t * NW < nchunks)

        def chunk_off(t):
            return (wid + t * NW) * CB

        def issue_idx(t, z):
            @pl.when(guard(t))
            def _():
                off = chunk_off(t)
                pltpu.async_copy(row_h.at[pl.ds(off, CB)], idxa[z], i_sem[z])
                pltpu.async_copy(col_h.at[pl.ds(off, CB)], idxb[z], i_sem[z])

        def start_gather(t, z):
            @pl.when(guard(t))
            def _():
                pltpu.make_async_copy(row_h.at[pl.ds(0, CB)], idxa[z],
                                      i_sem[z]).wait()
                pltpu.make_async_copy(col_h.at[pl.ds(0, CB)], idxb[z],
                                      i_sem[z]).wait()
                pltpu.async_copy(a_h.at[idxa[z]], bufa[z], g_sem[z])
                pltpu.async_copy(b_h.at[idxb[z]], bufb[z], g_sem[z])
                i16 = _iota16()
                lane0 = jnp.zeros((16,), jnp.int32)
                for g in range(CB // 16):
                    rv = idxa[z][pl.ds(g * 16, 16)]
                    cv = idxb[z][pl.ds(g * 16, 16)]
                    r4 = rv * 3
                    c4 = cv * 3
                    rad = None
                    for comp in range(3):
                        xr = plsc.load_gather(xn_v, [r4 + comp])
                        xc = plsc.load_gather(xn_v, [c4 + comp])
                        dx = xr - xc
                        plsc.store_scatter(rbuf[z], [g * 16 + i16,
                                                     lane0 + 1 + comp], dx)
                        rad = dx * dx if rad is None else rad + dx * dx
                    plsc.store_scatter(rbuf[z], [g * 16 + i16, lane0], rad)

        def start_write(t, z):
            @pl.when(guard(t))
            def _():
                off = chunk_off(t)
                pltpu.make_async_copy(a_h.at[idxa[z]], bufa[z],
                                      g_sem[z]).wait()
                pltpu.make_async_copy(b_h.at[idxb[z]], bufb[z],
                                      g_sem[z]).wait()
                pltpu.async_copy(bufa[z], ga_h.at[pl.ds(off, CB)], w_sem[z])
                pltpu.async_copy(bufb[z], gb_h.at[pl.ds(off, CB)], w_sem[z])
                pltpu.async_copy(rbuf[z],
                                 r16_h.at[pl.ds(off, CB), pl.ds(0, 16)],
                                 w_sem[z])

        def wait_write(t, z):
            @pl.when(guard(t))
            def _():
                off = chunk_off(t)
                pltpu.make_async_copy(bufa[z], ga_h.at[pl.ds(off, CB)],
                                      w_sem[z]).wait()
                pltpu.make_async_copy(bufb[z], gb_h.at[pl.ds(off, CB)],
                                      w_sem[z]).wait()
                pltpu.make_async_copy(rbuf[z],
                                      r16_h.at[pl.ds(off, CB),
                                               pl.ds(0, 16)],
                                      w_sem[z]).wait()

        def body(u, carry):
            for z in range(KSLOT):
                t = u * KSLOT + z
                wait_write(t - KSLOT, z)
                issue_idx(t, z)
                start_gather(t - 1, (z - 1) % KSLOT)
                start_write(t - 2, (z - 2) % KSLOT)
            return carry

        lax.fori_loop(0, nu, body, 0)
        for c in range(max(nu * KSLOT - KSLOT, 0), cpw):
            wait_write(jnp.int32(c), c % KSLOT)

    return k(row, col, a_t, b_t, xn1d)


def _sc_scatter(row, s, t16, zrows, zrows16):
    ne = row.shape[0]
    nchunks = ne // CB
    cpw = -(-nchunks // NW)
    mesh = plsc.VectorSubcoreMesh(core_axis_name="c", subcore_axis_name="s")
    nsteps = cpw + 2
    nu = -(-nsteps // KSLOT)

    @functools.partial(
        pl.kernel,
        mesh=mesh,
        out_type=[jax.ShapeDtypeStruct((NPAD, HID), _f32),
                  jax.ShapeDtypeStruct((NPAD, HID), _f32),
                  jax.ShapeDtypeStruct((NPAD, 16), _f32),
                  jax.ShapeDtypeStruct((NPAD, 16), _f32)],
        scratch_types=([pltpu.VMEM((CB,), jnp.int32)] * KSLOT
                       + [pltpu.VMEM((CB, HID), _f32)] * KSLOT
                       + [pltpu.VMEM((CB, 16), _f32)] * KSLOT
                       + [pltpu.VMEM_SHARED((NPAD, HID), _f32)]
                       + [pltpu.VMEM_SHARED((NPAD, 16), _f32)]
                       + [pltpu.SemaphoreType.DMA] * (3 * KSLOT)),
        compiler_params=pltpu.CompilerParams(needs_layout_passes=False, use_tc_tiling_on_sc=False),
    )
    def k(row_h, s_h, t_h, z_h, z16_h, p0_h, p1_h, q0_h, q1_h, *scr):
        idx = scr[0:KSLOT]
        buf = scr[KSLOT:2 * KSLOT]
        tbuf = scr[2 * KSLOT:3 * KSLOT]
        acc = scr[3 * KSLOT]
        acc2 = scr[3 * KSLOT + 1]
        l_sem = scr[3 * KSLOT + 2:4 * KSLOT + 2]
        a_sem = scr[4 * KSLOT + 2:5 * KSLOT + 2]
        t_sem = scr[5 * KSLOT + 2:6 * KSLOT + 2]
        cid = lax.axis_index("c")
        sid = lax.axis_index("s")
        wid = sid * NC + cid
        rbase = sid * ROWS_PER_SUB

        pltpu.sync_copy(z_h, acc.at[pl.ds(rbase, ROWS_PER_SUB)])
        pltpu.sync_copy(z16_h, acc2.at[pl.ds(rbase, ROWS_PER_SUB)])
        plsc.subcore_barrier()

        def guard(t):
            return (t >= 0) & (wid + t * NW < nchunks)

        def chunk_off(t):
            return (wid + t * NW) * CB

        def issue_load(t, z):
            @pl.when(guard(t))
            def _():
                off = chunk_off(t)
                pltpu.async_copy(row_h.at[pl.ds(off, CB)], idx[z], l_sem[z])
                pltpu.async_copy(s_h.at[pl.ds(off, CB)], buf[z], l_sem[z])
                pltpu.async_copy(t_h.at[pl.ds(off, CB), pl.ds(0, 16)],
                                 tbuf[z], l_sem[z])

        def start_add(t, z):
            @pl.when(guard(t))
            def _():
                pltpu.make_async_copy(row_h.at[pl.ds(0, CB)], idx[z],
                                      l_sem[z]).wait()
                pltpu.make_async_copy(s_h.at[pl.ds(0, CB)], buf[z],
                                      l_sem[z]).wait()
                pltpu.make_async_copy(t_h.at[pl.ds(0, CB), pl.ds(0, 16)],
                                      tbuf[z], l_sem[z]).wait()
                pltpu.async_copy(buf[z], acc.at[idx[z]], a_sem[z], add=True)
                pltpu.async_copy(tbuf[z], acc2.at[idx[z]], t_sem[z],
                                 add=True)

        def wait_add(t, z):
            @pl.when(guard(t))
            def _():
                pltpu.make_async_copy(buf[z], acc.at[idx[z]],
                                      a_sem[z]).wait()
                pltpu.make_async_copy(tbuf[z], acc2.at[idx[z]],
                                      t_sem[z]).wait()

        def body(u, carry):
            for z in range(KSLOT):
                t = u * KSLOT + z
                wait_add(t - KSLOT, z)
                issue_load(t, z)
                start_add(t - 1, (z - 1) % KSLOT)
            return carry

        lax.fori_loop(0, nu, body, 0)
        for c in range(max(nu * KSLOT - KSLOT, 0), cpw):
            wait_add(jnp.int32(c), c % KSLOT)
        plsc.subcore_barrier()

        @pl.when(cid == 0)
        def _():
            pltpu.sync_copy(acc.at[pl.ds(rbase, ROWS_PER_SUB)],
                            p0_h.at[pl.ds(rbase, ROWS_PER_SUB)])
            pltpu.sync_copy(acc2.at[pl.ds(rbase, ROWS_PER_SUB)],
                            q0_h.at[pl.ds(rbase, ROWS_PER_SUB)])

        @pl.when(cid == 1)
        def _():
            pltpu.sync_copy(acc.at[pl.ds(rbase, ROWS_PER_SUB)],
                            p1_h.at[pl.ds(rbase, ROWS_PER_SUB)])
            pltpu.sync_copy(acc2.at[pl.ds(rbase, ROWS_PER_SUB)],
                            q1_h.at[pl.ds(rbase, ROWS_PER_SUB)])

    return k(row, s, t16, zrows, zrows16)


def _xnorm(x4):
    def body(x_ref, o_ref):
        xv = x_ref[...]
        xs = xv - jnp.min(xv, axis=0, keepdims=True)
        xn = xs / jnp.max(xs, axis=0, keepdims=True)
        lane = lax.broadcasted_iota(jnp.int32, xn.shape, 1)
        o_ref[...] = jnp.where(lane < 3, xn, 0.0)

    return pl.pallas_call(
        body, out_shape=jax.ShapeDtypeStruct((N, 4), _f32))(x4)


def _emb(h0, w, b):
    def body(h_ref, w_ref, b_ref, o_ref):
        o_ref[...] = jnp.dot(h_ref[...], w_ref[...],
                             preferred_element_type=_f32) + b_ref[...]

    return pl.pallas_call(
        body,
        grid=(N // NBLK,),
        in_specs=[pl.BlockSpec((NBLK, HID), lambda i: (i, 0)),
                  pl.BlockSpec((HID, HID), lambda i: (0, 0)),
                  pl.BlockSpec((1, HID), lambda i: (0, 0))],
        out_specs=pl.BlockSpec((NBLK, HID), lambda i: (i, 0)),
        out_shape=jax.ShapeDtypeStruct((N, HID), _f32),
    )(h0, w, b)


def _prep(h, w1a, w1b, b1):
    def body(h_ref, wa_ref, wb_ref, b_ref, a_ref, bx_ref):
        hb = h_ref[...]
        a_ref[...] = jnp.dot(hb, wa_ref[...],
                             preferred_element_type=_f32) + b_ref[...]
        bx_ref[...] = jnp.dot(hb, wb_ref[...], preferred_element_type=_f32)

    return pl.pallas_call(
        body,
        grid=(N // NBLK,),
        in_specs=[pl.BlockSpec((NBLK, HID), lambda i: (i, 0)),
                  pl.BlockSpec((HID, HID), lambda i: (0, 0)),
                  pl.BlockSpec((HID, HID), lambda i: (0, 0)),
                  pl.BlockSpec((1, HID), lambda i: (0, 0))],
        out_specs=[pl.BlockSpec((NBLK, HID), lambda i: (i, 0)),
                   pl.BlockSpec((NBLK, HID), lambda i: (i, 0))],
        out_shape=[jax.ShapeDtypeStruct((N, HID), _f32),
                   jax.ShapeDtypeStruct((N, HID), _f32)],
    )(h, w1a, w1b, b1)


def _edge(ga, gb, r16, ea, w1, b1, w2, b2, wc1, bc1, wc2):
    def body(ga_ref, gb_ref, r_ref, ea_ref, w1_ref, b1_ref, w2_ref, b2_ref,
             wc1_ref, bc1_ref, wc2_ref, s_ref, cm_ref):
        radial = r_ref[...][:, :1]
        ein = jnp.concatenate(
            [ga_ref[...], gb_ref[...], radial, ea_ref[...]], axis=1)
        pre = jnp.dot(ein, w1_ref[...],
                      preferred_element_type=_f32) + b1_ref[...]
        u = _silu(pre)
        ef = _silu(jnp.dot(u, w2_ref[...], preferred_element_type=_f32)
                   + b2_ref[...])
        v = _silu(jnp.dot(ef, wc1_ref[...], preferred_element_type=_f32)
                  + bc1_ref[...])
        cm = jnp.dot(v, wc2_ref[...], preferred_element_type=_f32)
        s_ref[...] = ef
        trans = r_ref[...][:, 1:4] * cm[:, :1]
        cm_ref[...] = jnp.concatenate(
            [trans, jnp.zeros((trans.shape[0], HID - 3), _f32)], axis=1)

    ne = ga.shape[0]
    return pl.pallas_call(
        body,
        grid=(ne // EBLK,),
        in_specs=[pl.BlockSpec((EBLK, HID), lambda i: (i, 0)),
                  pl.BlockSpec((EBLK, HID), lambda i: (i, 0)),
                  pl.BlockSpec((EBLK, HID), lambda i: (i, 0)),
                  pl.BlockSpec((EBLK, 16), lambda i: (i, 0)),
                  pl.BlockSpec((273, HID), lambda i: (0, 0)),
                  pl.BlockSpec((1, HID), lambda i: (0, 0)),
                  pl.BlockSpec((HID, HID), lambda i: (0, 0)),
                  pl.BlockSpec((1, HID), lambda i: (0, 0)),
                  pl.BlockSpec((HID, HID), lambda i: (0, 0)),
                  pl.BlockSpec((1, HID), lambda i: (0, 0)),
                  pl.BlockSpec((HID, 8), lambda i: (0, 0))],
        out_specs=[pl.BlockSpec((EBLK, HID), lambda i: (i, 0)),
                   pl.BlockSpec((EBLK, HID), lambda i: (i, 0))],
        out_shape=[jax.ShapeDtypeStruct((ne, HID), _f32),
                   jax.ShapeDtypeStruct((ne, HID), _f32)],
    )(ga, gb, r16, ea, w1, b1, w2, b2, wc1, bc1, wc2)


def _node(h, h0, ps, qs, xn, xw, wn1, bn1, wn2, bn2):
    nps = len(ps)

    def body(h_ref, h0_ref, *rest):
        p_refs = rest[0:nps]
        q_refs = rest[nps:2 * nps]
        (xn_ref, xw_ref, wn1_ref, bn1_ref, wn2_ref, bn2_ref,
         hn_ref, xo_ref) = rest[2 * nps:]
        hb = h_ref[...]
        nagg = p_refs[0][...]
        for pr in p_refs[1:]:
            nagg = nagg + pr[...]
        agg = q_refs[0][...][:, :4]
        for qr in q_refs[1:]:
            agg = agg + qr[...][:, :4]
        nin = jnp.concatenate([hb, nagg, h0_ref[...]], axis=1)
        t = jnp.dot(nin, wn1_ref[...],
                    preferred_element_type=_f32) + bn1_ref[...]
        hn_ref[...] = hb + jnp.dot(_silu(t), wn2_ref[...],
                                   preferred_element_type=_f32) + bn2_ref[...]
        xo_ref[...] = xn_ref[...] + agg / xw_ref[...]

    return pl.pallas_call(
        body,
        grid=(N // NBLK,),
        in_specs=([pl.BlockSpec((NBLK, HID), lambda i: (i, 0))] * 2
                  + [pl.BlockSpec((NBLK, HID), lambda i: (i, 0))] * nps
                  + [pl.BlockSpec((NBLK, 16), lambda i: (i, 0))] * nps
                  + [pl.BlockSpec((NBLK, 4), lambda i: (i, 0)),
                     pl.BlockSpec((NBLK, 1), lambda i: (i, 0)),
                     pl.BlockSpec((3 * HID, HID), lambda i: (0, 0)),
                     pl.BlockSpec((1, HID), lambda i: (0, 0)),
                     pl.BlockSpec((HID, HID), lambda i: (0, 0)),
                     pl.BlockSpec((1, HID), lambda i: (0, 0))]),
        out_specs=[pl.BlockSpec((NBLK, HID), lambda i: (i, 0)),
                   pl.BlockSpec((NBLK, 4), lambda i: (i, 0))],
        out_shape=[jax.ShapeDtypeStruct((N, HID), _f32),
                   jax.ShapeDtypeStruct((N, 4), _f32)],
    )(h, h0, *ps, *qs, xn, xw, wn1, bn1, wn2, bn2)


def _dec(h, wd1, bd1, wd2, bd2, wg1, bg1, wg2, bg2):
    def body(h_ref, wd1_ref, bd1_ref, wd2_ref, bd2_ref,
             wg1_ref, bg1_ref, wg2_ref, bg2_ref, o_ref):
        t = _silu(jnp.dot(h_ref[...], wd1_ref[...],
                          preferred_element_type=_f32) + bd1_ref[...])
        t = jnp.dot(t, wd2_ref[...], preferred_element_type=_f32) + bd2_ref[...]
        u = _silu(jnp.dot(t, wg1_ref[...],
                          preferred_element_type=_f32) + bg1_ref[...])
        o_ref[...] = jnp.dot(u, wg2_ref[...],
                             preferred_element_type=_f32) + bg2_ref[...]

    return pl.pallas_call(
        body,
        grid=(N // NBLK,),
        in_specs=[pl.BlockSpec((NBLK, HID), lambda i: (i, 0)),
                  pl.BlockSpec((HID, HID), lambda i: (0, 0)),
                  pl.BlockSpec((1, HID), lambda i: (0, 0)),
                  pl.BlockSpec((HID, HID), lambda i: (0, 0)),
                  pl.BlockSpec((1, HID), lambda i: (0, 0)),
                  pl.BlockSpec((HID, HID), lambda i: (0, 0)),
                  pl.BlockSpec((1, HID), lambda i: (0, 0)),
                  pl.BlockSpec((HID, 21), lambda i: (0, 0)),
                  pl.BlockSpec((1, 21), lambda i: (0, 0))],
        out_specs=pl.BlockSpec((NBLK, 21), lambda i: (i, 0)),
        out_shape=jax.ShapeDtypeStruct((N, 21), _f32),
    )(h, wd1, bd1, wd2, bd2, wg1, bg1, wg2, bg2)


def kernel(h0, x, edges, edge_attr, x_weights, params):
    row = edges[0]
    col = edges[1]
    x4 = jnp.concatenate([x, jnp.zeros((N, 1), _f32)], axis=1)
    zrows = jnp.zeros((ROWS_PER_SUB, HID), _f32)
    zrows16 = jnp.zeros((ROWS_PER_SUB, 16), _f32)

    wemb, bemb = params['emb']
    h = _emb(h0, wemb, bemb.reshape(1, HID))

    for lp in params['layers']:
        W1, b1 = lp['edge_mlp'][0]
        W2, b2 = lp['edge_mlp'][1]
        Wc1, bc1 = lp['coord_mlp'][0]
        Wc2, _ = lp['coord_mlp'][1]
        Wn1, bn1 = lp['node_mlp'][0]
        Wn2, bn2 = lp['node_mlp'][1]
        wc2p = jnp.pad(Wc2, ((0, 0), (0, 7)))

        xn = _xnorm(x4)
        xn1d = jnp.ravel(xn[:, :3])
        ps, qs = [], []
        for hh in range(NHALF):
            lo = hh * EH
            rowh = lax.dynamic_slice_in_dim(row, lo, EH)
            colh = lax.dynamic_slice_in_dim(col, lo, EH)
            eah = lax.dynamic_slice_in_dim(edge_attr, lo, EH)
            ga, gb, r16 = _sc_gather(rowh, colh, h, h, xn1d)
            sh, t16 = _edge(ga, gb, r16, eah, W1, b1.reshape(1, HID), W2,
                            b2.reshape(1, HID), Wc1, bc1.reshape(1, HID),
                            wc2p)
            p0, p1, q0, q1 = _sc_scatter(rowh, sh, t16, zrows, zrows16)
            ps += [p0, p1]
            qs += [q0, q1]
        h, x4 = _node(h, h0, ps, qs, xn, x_weights,
                      Wn1, bn1.reshape(1, HID), Wn2, bn2.reshape(1, HID))

    Wd1, bd1 = params['node_dec'][0]
    Wd2, bd2 = params['node_dec'][1]
    Wg1, bg1 = params['graph_dec'][0]
    Wg2, bg2 = params['graph_dec'][1]
    return _dec(h, Wd1, bd1.reshape(1, HID), Wd2, bd2.reshape(1, HID),
                Wg1, bg1.reshape(1, HID), Wg2, bg2.reshape(1, 21))
